# baseline (speedup 1.0000x reference)
import functools
import jax
import jax.numpy as jnp
from jax import lax
from jax.experimental import pallas as pl
from jax.experimental.pallas import tpu as pltpu
from jax.experimental.pallas import tpu_sc as plsc

H = 128
LANES = 16
HV = H // LANES
NC = 2
NS = 16
NW = NC * NS
C = 128
NBUF = 3
LBLK = 512


def _sc_gather_pool(tgt_ids2d, msa_ids2d, table, B, L, n_msa):
    total = B * L
    P = total // NW
    n_chunks = P // C
    n_steps = n_chunks * n_msa
    inv_n = 1.0 / n_msa

    mesh = plsc.VectorSubcoreMesh(core_axis_name="c", subcore_axis_name="s")

    @functools.partial(
        pl.kernel,
        out_type=(
            jax.ShapeDtypeStruct((total, H), jnp.float32),
            jax.ShapeDtypeStruct((total, H), jnp.float32),
        ),
        mesh=mesh,
        scratch_types=[
            pltpu.VMEM((B, P), jnp.int32),
            pltpu.VMEM((n_chunks, n_msa, C), jnp.int32),
            pltpu.VMEM((P, H), jnp.float32),
            pltpu.VMEM((NBUF, C, H), jnp.float32),
            pltpu.VMEM((P, H), jnp.float32),
            pltpu.SemaphoreType.DMA,
            pltpu.SemaphoreType.DMA,
            pltpu.SemaphoreType.DMA,
            pltpu.SemaphoreType.DMA,
            pltpu.SemaphoreType.DMA,
            pltpu.SemaphoreType.DMA,
            pltpu.SemaphoreType.DMA,
        ],
    )
    def k(tgt_hbm, msa_hbm, table_hbm, s_out, m_out,
          tidx, midx, trows, ring, acc,
          sem_i, sem_t, sem_a, sem_b0, sem_b1, sem_b2, sem_o):
        wid = lax.axis_index("s") * NC + lax.axis_index("c")
        base = wid * P
        b = base // L
        l0 = base - b * L
        bsems = (sem_b0, sem_b1, sem_b2)

        icps = [pltpu.make_async_copy(
            tgt_hbm.at[:, pl.ds(l0, P)], tidx, sem_i)]
        for ci in range(n_chunks):
            icps.append(pltpu.make_async_copy(
                msa_hbm.at[pl.ds(b * n_msa, n_msa), pl.ds(l0 + ci * C, C)],
                midx.at[ci], sem_i))
        for cp in icps:
            cp.start()
        for cp in icps:
            cp.wait()

        tcps = []
        for ci in range(n_chunks):
            cp = pltpu.make_async_copy(
                table_hbm.at[tidx.at[b, pl.ds(ci * C, C)]],
                trows.at[pl.ds(ci * C, C)], sem_t)
            cp.start()
            tcps.append(cp)

        ocps = []

        def fire(step):
            ci, j = divmod(step, n_msa)
            isl = midx.at[ci, j]
            if j == 0:
                cp = pltpu.make_async_copy(
                    table_hbm.at[isl], acc.at[pl.ds(ci * C, C)], sem_a)
            else:
                sl = step % NBUF
                cp = pltpu.make_async_copy(
                    table_hbm.at[isl], ring.at[sl], bsems[sl])
            cp.start()
            return cp

        def finalize(ci):
            tcps[ci].wait()
            a0 = ci * C

            def fin_row(p, _):
                r = a0 + p
                for h in range(HV):
                    hs = pl.ds(h * LANES, LANES)
                    m = acc[r, hs] * inv_n
                    acc[r, hs] = m
                    plsc.addupdate(trows.at[r, hs], m)
                return 0

            lax.fori_loop(0, C, fin_row, 0)
            for ref, out in ((trows, s_out), (acc, m_out)):
                cp = pltpu.make_async_copy(
                    ref.at[pl.ds(a0, C)], out.at[pl.ds(base + a0, C)], sem_o)
                cp.start()
                ocps.append(cp)

        cps = {s: fire(s) for s in range(min(NBUF, n_steps))}
        for step in range(n_steps):
            cps.pop(step).wait()
            ci, j = divmod(step, n_msa)
            if j > 0:
                src = ring.at[step % NBUF]
                a0 = ci * C

                def add_rows(i, _):
                    p = i * 2
                    for dp in range(2):
                        for h in range(HV):
                            hs = pl.ds(h * LANES, LANES)
                            plsc.addupdate(acc.at[a0 + p + dp, hs],
                                           src[p + dp, hs])
                    return 0

                lax.fori_loop(0, C // 2, add_rows, 0)
            if step + NBUF < n_steps:
                cps[step + NBUF] = fire(step + NBUF)
            if j == n_msa - 1:
                finalize(ci)

        for cp in ocps:
            cp.wait()

    return k(tgt_ids2d, msa_ids2d, table)


def _ln_body(s_ref, pos_ref, gamma_ref, beta_ref, out_ref):
    x = s_ref[0] + pos_ref[...]
    mean = jnp.mean(x, axis=-1, keepdims=True)
    cx = x - mean
    var = jnp.mean(cx * cx, axis=-1, keepdims=True)
    inv = lax.rsqrt(var + 1e-12)
    out_ref[0] = cx * inv * gamma_ref[0] + beta_ref[0]


def kernel(target_ids, input_ids, word_embeddings, position_embeddings, gamma, beta):
    B, L = target_ids.shape
    n_msa = input_ids.shape[1]
    tgt_idx = target_ids.astype(jnp.int32)
    msa_idx = input_ids.astype(jnp.int32).reshape(B * n_msa, L)
    s, msa_mean = _sc_gather_pool(tgt_idx, msa_idx, word_embeddings, B, L, n_msa)
    emb = pl.pallas_call(
        _ln_body,
        grid=(B, L // LBLK),
        in_specs=[
            pl.BlockSpec((1, LBLK, H), lambda b, l: (b, l, 0)),
            pl.BlockSpec((LBLK, H), lambda b, l: (l, 0)),
            pl.BlockSpec((1, H), lambda b, l: (0, 0)),
            pl.BlockSpec((1, H), lambda b, l: (0, 0)),
        ],
        out_specs=pl.BlockSpec((1, LBLK, H), lambda b, l: (b, l, 0)),
        out_shape=jax.ShapeDtypeStruct((B, L, H), jnp.float32),
        compiler_params=pltpu.CompilerParams(
            dimension_semantics=("parallel", "arbitrary")),
    )(s.reshape(B, L, H), position_embeddings, gamma.reshape(1, H),
      beta.reshape(1, H))
    return emb, msa_mean.reshape(B, L, H)


# speedup vs baseline: 1.1336x; 1.1336x over previous
import functools
import jax
import jax.numpy as jnp
from jax import lax
from jax.experimental import pallas as pl
from jax.experimental.pallas import tpu as pltpu
from jax.experimental.pallas import tpu_sc as plsc

H = 128
LANES = 16
HV = H // LANES
NC = 2
NS = 16
NW = NC * NS
C = 128
NBUF = 3
LBLK = 512


def _sc_gather_pool(tgt_ids2d, msa_ids2d, table, B, L, n_msa):
    total = B * L
    P = total // NW
    n_chunks = P // C
    n_steps = n_chunks * n_msa
    inv_n = 1.0 / n_msa

    mesh = plsc.VectorSubcoreMesh(core_axis_name="c", subcore_axis_name="s")

    @functools.partial(
        pl.kernel,
        out_type=(
            jax.ShapeDtypeStruct((total, H), jnp.float32),
            jax.ShapeDtypeStruct((total, H), jnp.float32),
        ),
        mesh=mesh,
        scratch_types=[
            pltpu.VMEM((B, P), jnp.int32),
            pltpu.VMEM((n_chunks, n_msa, C), jnp.int32),
            pltpu.VMEM((P, H), jnp.float32),
            pltpu.VMEM((NBUF, C, H), jnp.float32),
            pltpu.VMEM((P, H), jnp.float32),
            pltpu.SemaphoreType.DMA,
            pltpu.SemaphoreType.DMA,
            pltpu.SemaphoreType.DMA,
            pltpu.SemaphoreType.DMA,
            pltpu.SemaphoreType.DMA,
            pltpu.SemaphoreType.DMA,
            pltpu.SemaphoreType.DMA,
        ],
    )
    def k(tgt_hbm, msa_hbm, table_hbm, s_out, m_out,
          tidx, midx, trows, ring, acc,
          sem_i, sem_t, sem_a, sem_b0, sem_b1, sem_b2, sem_o):
        wid = lax.axis_index("s") * NC + lax.axis_index("c")
        base = wid * P
        b = base // L
        l0 = base - b * L
        bsems = (sem_b0, sem_b1, sem_b2)

        icps = [pltpu.make_async_copy(
            tgt_hbm.at[:, pl.ds(l0, P)], tidx, sem_i)]
        for ci in range(n_chunks):
            icps.append(pltpu.make_async_copy(
                msa_hbm.at[pl.ds(b * n_msa, n_msa), pl.ds(l0 + ci * C, C)],
                midx.at[ci], sem_i))
        for cp in icps:
            cp.start()
        for cp in icps:
            cp.wait()

        tcps = []
        for ci in range(n_chunks):
            cp = pltpu.make_async_copy(
                table_hbm.at[tidx.at[b, pl.ds(ci * C, C)]],
                trows.at[pl.ds(ci * C, C)], sem_t)
            cp.start()
            tcps.append(cp)

        ocps = []

        def fire(step):
            ci, j = divmod(step, n_msa)
            isl = midx.at[ci, j]
            if j == 0:
                cp = pltpu.make_async_copy(
                    table_hbm.at[isl], acc.at[pl.ds(ci * C, C)], sem_a)
            else:
                sl = step % NBUF
                cp = pltpu.make_async_copy(
                    table_hbm.at[isl], ring.at[sl], bsems[sl])
            cp.start()
            return cp

        def finalize(ci):
            tcps[ci].wait()
            a0 = ci * C

            def fin_row(p, _):
                r = a0 + p
                for h in range(HV):
                    hs = pl.ds(h * LANES, LANES)
                    m = acc[r, hs] * inv_n
                    acc[r, hs] = m
                    plsc.addupdate(trows.at[r, hs], m)
                return 0

            lax.fori_loop(0, C, fin_row, 0)
            for ref, out in ((trows, s_out), (acc, m_out)):
                cp = pltpu.make_async_copy(
                    ref.at[pl.ds(a0, C)], out.at[pl.ds(base + a0, C)], sem_o)
                cp.start()
                ocps.append(cp)

        cps = {s: fire(s) for s in range(min(NBUF, n_steps))}
        for step in range(n_steps):
            cps.pop(step).wait()
            ci, j = divmod(step, n_msa)
            if j > 0:
                src = ring.at[step % NBUF]
                a0 = ci * C

                def add_rows(i, _):
                    p = i * 2
                    for dp in range(2):
                        for h in range(HV):
                            hs = pl.ds(h * LANES, LANES)
                            plsc.addupdate(acc.at[a0 + p + dp, hs],
                                           src[p + dp, hs])
                    return 0

                lax.fori_loop(0, C // 2, add_rows, 0)
            if step + NBUF < n_steps:
                cps[step + NBUF] = fire(step + NBUF)
            if j == n_msa - 1:
                finalize(ci)

        for cp in ocps:
            cp.wait()

    return k(tgt_ids2d, msa_ids2d, table)


def _ln_body(s_ref, pos_ref, gamma_ref, beta_ref, out_ref):
    x = s_ref[0] + pos_ref[...]
    mean = jnp.mean(x, axis=-1, keepdims=True)
    cx = x - mean
    var = jnp.mean(cx * cx, axis=-1, keepdims=True)
    inv = lax.rsqrt(var + 1e-12)
    out_ref[0] = cx * inv * gamma_ref[0] + beta_ref[0]


def kernel(target_ids, input_ids, word_embeddings, position_embeddings, gamma, beta):
    B, L = target_ids.shape
    n_msa = input_ids.shape[1]
    tgt_idx = target_ids.astype(jnp.int32)
    msa_idx = input_ids.astype(jnp.int32).reshape(B * n_msa, L)
    s, msa_mean = _sc_gather_pool(tgt_idx, msa_idx, word_embeddings, B, L, n_msa)
    emb = pl.pallas_call(
        _ln_body,
        grid=(B,),
        in_specs=[
            pl.BlockSpec((1, L, H), lambda b: (b, 0, 0)),
            pl.BlockSpec((L, H), lambda b: (0, 0)),
            pl.BlockSpec((1, H), lambda b: (0, 0)),
            pl.BlockSpec((1, H), lambda b: (0, 0)),
        ],
        out_specs=pl.BlockSpec((1, L, H), lambda b: (b, 0, 0)),
        out_shape=jax.ShapeDtypeStruct((B, L, H), jnp.float32),
    )(s.reshape(B, L, H), position_embeddings, gamma.reshape(1, H),
      beta.reshape(1, H))
    return emb, msa_mean.reshape(B, L, H)
